# baseline (speedup 1.0000x reference)
import functools

import jax
import jax.numpy as jnp
from jax import lax
from jax.scipy.linalg import block_diag
from jax.experimental import pallas as pl
from jax.experimental.pallas import tpu as pltpu
from jax.experimental.pallas import tpu_sc as plsc

N = 100000
E = 1600000
R = 5
NB = 2
NT = 100352
PW = 6272
CH = 128
NCH = 784
EPT = NCH * CH
EPAD = 16 * EPT
NBUF = 8
NSP = NT + 512
F32 = jnp.float32
I32 = jnp.int32


def _mesh():
    return plsc.VectorSubcoreMesh(core_axis_name="c", subcore_axis_name="s")


_SC_PARAMS = pltpu.CompilerParams(use_tc_tiling_on_sc=False,
                                  needs_layout_passes=False)


@functools.lru_cache(maxsize=None)
def _tc_layer0():
    Bn = 896

    def body(x_ref, wd_ref, lw_ref, b_ref, tab_ref, hl_ref):
        h = x_ref[...]
        hp = jnp.concatenate([h[q * 112:(q + 1) * 112] for q in range(8)],
                             axis=1)
        for r in range(R):
            tab_ref[0, r] = jnp.dot(hp, wd_ref[r, 0], preferred_element_type=F32)
            tab_ref[1, r] = jnp.dot(hp, wd_ref[r, 1], preferred_element_type=F32)
        hl_ref[...] = jnp.dot(h, lw_ref[...], preferred_element_type=F32) + b_ref[...]

    return pl.pallas_call(
        body, grid=(NT // Bn,),
        in_specs=[
            pl.BlockSpec((Bn, 4), lambda i: (i, 0)),
            pl.BlockSpec((R, 2, 32, 128), lambda i: (0, 0, 0, 0)),
            pl.BlockSpec((4, 32), lambda i: (0, 0)),
            pl.BlockSpec((1, 32), lambda i: (0, 0)),
        ],
        out_specs=[
            pl.BlockSpec((2, R, Bn // 8, 128), lambda i: (0, 0, i, 0)),
            pl.BlockSpec((Bn, 32), lambda i: (i, 0)),
        ],
        out_shape=[
            jax.ShapeDtypeStruct((2, R, NT // 8, 128), F32),
            jax.ShapeDtypeStruct((NT, 32), F32),
        ],
    )


@functools.lru_cache(maxsize=None)
def _tc_layer():
    Bn = 896

    def body(alo_ref, ahi_ref, hlp_ref, wd_ref, lw_ref, b_ref,
             s_ref, tab_ref, hl_ref):
        alo = jnp.concatenate(
            [alo_ref[0][:, q * 16:(q + 1) * 16] for q in range(8)], axis=0)
        ahi = jnp.concatenate(
            [ahi_ref[0][:, q * 16:(q + 1) * 16] for q in range(8)], axis=0)
        agg = jnp.concatenate([alo, ahi], axis=-1)
        h = jnp.tanh(agg + hlp_ref[...])
        s_ref[...] = h
        hp = jnp.concatenate([h[q * 112:(q + 1) * 112] for q in range(8)],
                             axis=1)
        for r in range(R):
            tab_ref[0, r] = jnp.dot(hp, wd_ref[r, 0], preferred_element_type=F32)
            tab_ref[1, r] = jnp.dot(hp, wd_ref[r, 1], preferred_element_type=F32)
        hl_ref[...] = jnp.dot(h, lw_ref[...], preferred_element_type=F32) + b_ref[...]

    return pl.pallas_call(
        body, grid=(NT // Bn,),
        in_specs=[
            pl.BlockSpec((1, Bn // 8, 128), lambda i: (0, i, 0)),
            pl.BlockSpec((1, Bn // 8, 128), lambda i: (1, i, 0)),
            pl.BlockSpec((Bn, 32), lambda i: (i, 0)),
            pl.BlockSpec((R, 2, 256, 128), lambda i: (0, 0, 0, 0)),
            pl.BlockSpec((32, 32), lambda i: (0, 0)),
            pl.BlockSpec((1, 32), lambda i: (0, 0)),
        ],
        out_specs=[
            pl.BlockSpec((Bn, 32), lambda i: (i, 0)),
            pl.BlockSpec((2, R, Bn // 8, 128), lambda i: (0, 0, i, 0)),
            pl.BlockSpec((Bn, 32), lambda i: (i, 0)),
        ],
        out_shape=[
            jax.ShapeDtypeStruct((NT, 32), F32),
            jax.ShapeDtypeStruct((2, R, NT // 8, 128), F32),
            jax.ShapeDtypeStruct((NT, 32), F32),
        ],
    )


@functools.lru_cache(maxsize=None)
def _tc_final_ab():
    Bn = 896

    def body(alo_ref, ahi_ref, hl3_ref, s1_ref, s2_ref, s3_ref, wt_ref, wb_ref,
             ab_ref):
        alo = jnp.concatenate(
            [alo_ref[0][:, q * 16:(q + 1) * 16] for q in range(8)], axis=0)
        ahi = jnp.concatenate(
            [ahi_ref[0][:, q * 16:(q + 1) * 16] for q in range(8)], axis=0)
        agg = jnp.concatenate([alo, ahi], axis=-1)
        s4 = jnp.tanh(agg + hl3_ref[...])
        states = (s1_ref[...], s2_ref[...], s3_ref[...], s4)
        wt = wt_ref[...]
        wb = wb_ref[...]
        a = jnp.dot(states[0], wt[0:32, :], preferred_element_type=F32)
        b = jnp.dot(states[0], wb[0:32, :], preferred_element_type=F32)
        for k in range(1, 4):
            a = a + jnp.dot(states[k], wt[32 * k:32 * k + 32, :],
                            preferred_element_type=F32)
            b = b + jnp.dot(states[k], wb[32 * k:32 * k + 32, :],
                            preferred_element_type=F32)
        ab_ref[0] = a
        ab_ref[1] = b

    return pl.pallas_call(
        body, grid=(NT // Bn,),
        in_specs=[
            pl.BlockSpec((1, Bn // 8, 128), lambda i: (0, i, 0)),
            pl.BlockSpec((1, Bn // 8, 128), lambda i: (1, i, 0)),
            pl.BlockSpec((Bn, 32), lambda i: (i, 0)),
            pl.BlockSpec((Bn, 32), lambda i: (i, 0)),
            pl.BlockSpec((Bn, 32), lambda i: (i, 0)),
            pl.BlockSpec((Bn, 32), lambda i: (i, 0)),
            pl.BlockSpec((128, 128), lambda i: (0, 0)),
            pl.BlockSpec((128, 128), lambda i: (0, 0)),
        ],
        out_specs=[pl.BlockSpec((2, Bn, 128), lambda i: (0, i, 0))],
        out_shape=[jax.ShapeDtypeStruct((2, NT, 128), F32)],
    )


@functools.lru_cache(maxsize=None)
def _tc_out():
    Bn = 896

    def body(su_ref, si_ref, b1_ref, w2_ref, b2_ref, o_ref):
        z = jnp.maximum(su_ref[0] + si_ref[0] + b1_ref[...], 0.0)
        o_ref[...] = jnp.dot(z, w2_ref[...], preferred_element_type=F32) + b2_ref[...]

    return pl.pallas_call(
        body, grid=(NT // Bn,),
        in_specs=[
            pl.BlockSpec((1, Bn, 128), lambda i: (0, i, 0)),
            pl.BlockSpec((1, Bn, 128), lambda i: (1, i, 0)),
            pl.BlockSpec((1, 128), lambda i: (0, 0)),
            pl.BlockSpec((128, 1), lambda i: (0, 0)),
            pl.BlockSpec((1, 1), lambda i: (0, 0)),
        ],
        out_specs=[pl.BlockSpec((Bn, 1), lambda i: (i, 0))],
        out_shape=[jax.ShapeDtypeStruct((NT, 1), F32)],
    )


@functools.lru_cache(maxsize=None)
def _sc_edge():
    @functools.partial(
        pl.kernel, mesh=_mesh(), compiler_params=_SC_PARAMS,
        out_type=jax.ShapeDtypeStruct((2, NT, 16), F32),
        scratch_types=[
            [pltpu.VMEM((CH,), I32)] * NBUF,
            [pltpu.VMEM((CH,), I32)] * NBUF,
            [pltpu.VMEM((CH,), I32)] * NBUF,
            [pltpu.VMEM((CH,), I32)] * NBUF,
            [pltpu.VMEM((CH, 16), F32)] * NBUF,
            pltpu.VMEM((112, 16), F32),
            pltpu.VMEM_SHARED((NT, 16), F32),
            [pltpu.SemaphoreType.DMA] * NBUF,
            [pltpu.SemaphoreType.DMA] * NBUF,
        ],
    )
    def ek(tab_hbm, ei_hbm, out_hbm,
           srcb, dstb, gb, dstg, rows, zb, aggsp, semi, semg):
        c = lax.axis_index("c")
        s = lax.axis_index("s")
        coff = c * (R * NT)
        tbase = s * PW

        def zfill(i, _):
            zb[i] = jnp.zeros((16,), F32)
            return 0
        lax.fori_loop(0, 112, zfill, 0)

        def zcopy(z, _):
            pltpu.sync_copy(zb, aggsp.at[pl.ds(tbase + z * 112, 112)])
            return 0
        lax.fori_loop(0, PW // 112, zcopy, 0)
        plsc.subcore_barrier()

        ebase = s * EPT

        def idx_start(k, b):
            base = ebase + k * CH
            pltpu.async_copy(ei_hbm.at[0, pl.ds(base, CH)], srcb[b], semi[b])
            pltpu.async_copy(ei_hbm.at[1, pl.ds(base, CH)], dstb[b], semi[b])

        def idx_wait(b):
            pltpu.make_async_copy(ei_hbm.at[0, pl.ds(0, CH)], srcb[b], semi[b]).wait()
            pltpu.make_async_copy(ei_hbm.at[0, pl.ds(0, CH)], dstb[b], semi[b]).wait()

        def compute_g(b):
            for j in range(CH // 16):
                sl = pl.ds(j * 16, 16)
                gb[b][sl] = srcb[b][sl] + coff
                dstg[b][sl] = dstb[b][sl]

        def g_start(b):
            pltpu.async_copy(tab_hbm.at[gb[b]], rows[b], semg[b])

        def g_wait_scatter(b):
            pltpu.make_async_copy(tab_hbm.at[gb[b]], rows[b], semg[b]).wait()
            pltpu.sync_copy(rows[b], aggsp.at[dstg[b]], add=True)

        for k0 in range(NBUF):
            idx_start(k0, k0)
        for k0 in range(NBUF - 1):
            idx_wait(k0)
            compute_g(k0)
            g_start(k0)
            idx_start(k0 + NBUF, k0)

        def body(k4, _):
            for j in range(NBUF):
                b = (NBUF - 1 + j) % NBUF
                k = (NBUF - 1) + k4 * NBUF + j
                idx_wait(b)
                compute_g(b)
                g_start(b)
                idx_start(k + NBUF, b)
                g_wait_scatter((b + 1) % NBUF)
            return 0

        lax.fori_loop(0, (NCH - NBUF) // NBUF, body, 0)

        bl = (NCH - 1) % NBUF
        idx_wait(bl)
        compute_g(bl)
        g_start(bl)
        for d in range(NBUF):
            g_wait_scatter((bl + 1 + d) % NBUF)
        for d in range(NBUF - 1):
            idx_wait((bl + 1 + d) % NBUF)

        plsc.subcore_barrier()
        pltpu.sync_copy(aggsp.at[pl.ds(tbase, PW)],
                        out_hbm.at[c, pl.ds(tbase, PW)])

    return ek


@functools.lru_cache(maxsize=None)
def _sc_compact():
    ZW = NSP // 16

    @functools.partial(
        pl.kernel, mesh=_mesh(), compiler_params=_SC_PARAMS,
        out_type=jax.ShapeDtypeStruct((2, NSP), I32),
        scratch_types=[
            pltpu.VMEM((PW,), F32),
            pltpu.VMEM((128,), I32),
            pltpu.VMEM((128,), I32),
            pltpu.VMEM((16,), I32),
            pltpu.VMEM((256,), I32),
            pltpu.VMEM((ZW,), I32),
            pltpu.VMEM_SHARED((NSP,), I32),
            pltpu.VMEM_SHARED((256,), I32),
        ],
    )
    def ck(xu_hbm, xi_hbm, out_hbm,
           condb, posb, valb, cntb, callb, zbc, idxsp, cntsp):
        c = lax.axis_index("c")
        s = lax.axis_index("s")
        base = s * PW
        iota = lax.iota(I32, 16)

        @pl.when(c == 0)
        def _():
            pltpu.sync_copy(xu_hbm.at[pl.ds(base, PW)], condb)

        @pl.when(c == 1)
        def _():
            pltpu.sync_copy(xi_hbm.at[pl.ds(base, PW)], condb)

        def zfill(i, _):
            zbc[pl.ds(i * 16, 16)] = jnp.zeros((16,), I32)
            return 0
        lax.fori_loop(0, ZW // 16, zfill, 0)
        pltpu.sync_copy(zbc, idxsp.at[pl.ds(s * ZW, ZW)])

        one16 = jnp.ones((16,), I32)
        zero16 = jnp.zeros((16,), I32)

        def cnt(i, acc):
            f = condb[pl.ds(i * 16, 16)]
            return acc + jnp.sum(jnp.where(f != 0.0, one16, zero16))
        total = lax.fori_loop(0, PW // 16, cnt, jnp.zeros((), I32))
        cntb[...] = total + jnp.zeros((16,), I32)
        pltpu.sync_copy(cntb, cntsp.at[pl.ds(s * 16, 16)])
        plsc.subcore_barrier()

        pltpu.sync_copy(cntsp, callb)
        cvec = plsc.load_gather(callb, [iota * 16])
        excl0 = jnp.sum(jnp.where(iota < s, cvec, jnp.zeros((16,), I32)))

        def body(ko, off):
            for j in range(8):
                i = ko * 8 + j
                f = condb[pl.ds(i * 16, 16)]
                v = jnp.where(f != 0.0, one16, zero16)
                incl = plsc.cumsum(v)
                pos = off + incl - 1
                posm = jnp.where(v == 1, pos, NT + iota)
                sl = pl.ds(j * 16, 16)
                posb[sl] = posm
                valb[sl] = base + i * 16 + iota
                off = off + jnp.sum(v)
            pltpu.sync_copy(valb, idxsp.at[posb])
            return off

        lax.fori_loop(0, PW // 128, body, excl0)
        plsc.subcore_barrier()
        pltpu.sync_copy(idxsp.at[pl.ds(base, PW)], out_hbm.at[c, pl.ds(base, PW)])

        @pl.when(s == 15)
        def _():
            pltpu.sync_copy(idxsp.at[pl.ds(NT, 512)],
                            out_hbm.at[c, pl.ds(NT, 512)])

    return ck


@functools.lru_cache(maxsize=None)
def _sc_gather_ab():
    GC = 112
    NCHG = PW // GC
    NBG = 4

    @functools.partial(
        pl.kernel, mesh=_mesh(), compiler_params=_SC_PARAMS,
        out_type=jax.ShapeDtypeStruct((2, NT, 128), F32),
        scratch_types=[
            [pltpu.VMEM((GC,), I32)] * NBG,
            [pltpu.VMEM((GC,), I32)] * NBG,
            [pltpu.VMEM((GC, 128), F32)] * NBG,
            [pltpu.SemaphoreType.DMA] * NBG,
            [pltpu.SemaphoreType.DMA] * NBG,
        ],
    )
    def gk(uij_hbm, ab_hbm, out_hbm, idxb, gb, rows, semi, semg):
        c = lax.axis_index("c")
        s = lax.axis_index("s")
        base = s * PW
        coff = c * NT

        def idx_start(k, b):
            pltpu.async_copy(uij_hbm.at[c, pl.ds(base + k * GC, GC)],
                             idxb[b], semi[b])

        def idx_wait(b):
            pltpu.make_async_copy(uij_hbm.at[c, pl.ds(base, GC)],
                                  idxb[b], semi[b]).wait()

        def compute_g(b):
            for j in range(GC // 16):
                sl = pl.ds(j * 16, 16)
                gb[b][sl] = idxb[b][sl] + coff

        def g_start(b):
            pltpu.async_copy(ab_hbm.at[gb[b]], rows[b], semg[b])

        def g_wait_store(b, k):
            pltpu.make_async_copy(ab_hbm.at[gb[b]], rows[b], semg[b]).wait()
            pltpu.sync_copy(rows[b], out_hbm.at[c, pl.ds(base + k * GC, GC)])

        for k0 in range(NBG):
            idx_start(k0, k0)
        for k0 in range(NBG - 1):
            idx_wait(k0)
            compute_g(k0)
            g_start(k0)
            idx_start(k0 + NBG, k0)

        def body(k4, _):
            for j in range(NBG):
                b = (NBG - 1 + j) % NBG
                k = (NBG - 1) + k4 * NBG + j
                idx_wait(b)
                compute_g(b)
                g_start(b)
                idx_start(k + NBG, b)
                g_wait_store((b + 1) % NBG, k - NBG + 1)
            return 0

        lax.fori_loop(0, (NCHG - NBG) // NBG, body, 0)

        bl = (NCHG - 1) % NBG
        idx_wait(bl)
        compute_g(bl)
        g_start(bl)
        for d in range(NBG):
            bo = (bl + 1 + d) % NBG
            g_wait_store(bo, NCHG - NBG + d)
        for d in range(NBG - 1):
            idx_wait((bl + 1 + d) % NBG)

    return gk


def _wdiag(V, comp):
    din = V.shape[1]
    W3 = jnp.dot(comp, V.reshape(NB, din * 32)).reshape(R, din, 32)
    rows = []
    for r in range(R):
        rows.append(jnp.stack([
            block_diag(*([W3[r, :, h * 16:(h + 1) * 16]] * 8)) for h in (0, 1)
        ]))
    return jnp.stack(rows)


def kernel(x, edge_index, etype, V0, comp0, loop0, b0, V1, comp1, loop1, b1,
           V2, comp2, loop2, b2, V3, comp3, loop3, b3,
           lin1_w, lin1_b, lin2_w, lin2_b):
    npad = EPAD + NBUF * CH - E
    pad_src = jnp.zeros((1, npad), I32)
    pad_dst = jnp.full((1, npad), N, I32)
    eip = jnp.concatenate([edge_index, jnp.concatenate([pad_src, pad_dst], 0)], 1)
    etp = jnp.pad(etype, (0, npad))
    d1 = eip // 112
    rem = eip - d1 * 112
    eip = (d1 >> 3) * 896 + rem * 8 + (d1 & 7)
    eip = eip.at[0].add(etp * NT)
    xu = jnp.pad(x[:, 0], (0, NT - N))
    xi = jnp.pad(x[:, 1], (0, NT - N))
    xp = jnp.pad(x, ((0, NT - N), (0, 0)))

    l0 = _tc_layer0()
    ll = _tc_layer()
    ek = _sc_edge()
    tab0, hl0 = l0(xp, _wdiag(V0, comp0), loop0, b0.reshape(1, 32))
    agg0 = ek(tab0.reshape(2 * R * NT, 16), eip).reshape(2, NT // 8, 128)
    s1, tab1, hl1 = ll(agg0, agg0, hl0, _wdiag(V1, comp1), loop1, b1.reshape(1, 32))
    agg1 = ek(tab1.reshape(2 * R * NT, 16), eip).reshape(2, NT // 8, 128)
    s2, tab2, hl2 = ll(agg1, agg1, hl1, _wdiag(V2, comp2), loop2, b2.reshape(1, 32))
    agg2 = ek(tab2.reshape(2 * R * NT, 16), eip).reshape(2, NT // 8, 128)
    s3, tab3, hl3 = ll(agg2, agg2, hl2, _wdiag(V3, comp3), loop3, b3.reshape(1, 32))
    agg3 = ek(tab3.reshape(2 * R * NT, 16), eip).reshape(2, NT // 8, 128)

    ab, = _tc_final_ab()(agg3, agg3, hl3, s1, s2, s3,
                         lin1_w[:128, :], lin1_w[128:, :])
    uij = _sc_compact()(xu, xi)
    s2g = _sc_gather_ab()(uij, ab.reshape(2 * NT, 128))
    o, = _tc_out()(s2g, s2g, lin1_b.reshape(1, 128), lin2_w, lin2_b.reshape(1, 1))
    return o[:N, 0]


# speedup vs baseline: 1.0570x; 1.0570x over previous
import functools

import jax
import jax.numpy as jnp
from jax import lax
from jax.scipy.linalg import block_diag
from jax.experimental import pallas as pl
from jax.experimental.pallas import tpu as pltpu
from jax.experimental.pallas import tpu_sc as plsc

N = 100000
E = 1600000
R = 5
NB = 2
NT = 100352
PW = 6272
CH = 128
NCH = 784
EPT = NCH * CH
EPAD = 16 * EPT
NBUF = 8
NSP = NT + 512
F32 = jnp.float32
I32 = jnp.int32


def _mesh():
    return plsc.VectorSubcoreMesh(core_axis_name="c", subcore_axis_name="s")


_SC_PARAMS = pltpu.CompilerParams(use_tc_tiling_on_sc=False,
                                  needs_layout_passes=False)


@functools.lru_cache(maxsize=None)
def _tc_layer0():
    Bn = 896

    def body(x_ref, wd_ref, lw_ref, b_ref, tab_ref, hl_ref):
        h = x_ref[...]
        hp = jnp.concatenate([h[q * 112:(q + 1) * 112] for q in range(8)],
                             axis=1)
        for r in range(R):
            tab_ref[0, r] = jnp.dot(hp, wd_ref[r, 0], preferred_element_type=F32)
            tab_ref[1, r] = jnp.dot(hp, wd_ref[r, 1], preferred_element_type=F32)
        hl_ref[...] = jnp.dot(h, lw_ref[...], preferred_element_type=F32) + b_ref[...]

    return pl.pallas_call(
        body, grid=(NT // Bn,),
        in_specs=[
            pl.BlockSpec((Bn, 4), lambda i: (i, 0)),
            pl.BlockSpec((R, 2, 32, 128), lambda i: (0, 0, 0, 0)),
            pl.BlockSpec((4, 32), lambda i: (0, 0)),
            pl.BlockSpec((1, 32), lambda i: (0, 0)),
        ],
        out_specs=[
            pl.BlockSpec((2, R, Bn // 8, 128), lambda i: (0, 0, i, 0)),
            pl.BlockSpec((Bn, 32), lambda i: (i, 0)),
        ],
        out_shape=[
            jax.ShapeDtypeStruct((2, R, NT // 8, 128), F32),
            jax.ShapeDtypeStruct((NT, 32), F32),
        ],
    )


@functools.lru_cache(maxsize=None)
def _tc_layer():
    Bn = 896

    def body(alo_ref, ahi_ref, hlp_ref, wd_ref, lw_ref, b_ref,
             s_ref, tab_ref, hl_ref):
        alo = jnp.concatenate(
            [alo_ref[0][:, q * 16:(q + 1) * 16] for q in range(8)], axis=0)
        ahi = jnp.concatenate(
            [ahi_ref[0][:, q * 16:(q + 1) * 16] for q in range(8)], axis=0)
        agg = jnp.concatenate([alo, ahi], axis=-1)
        h = jnp.tanh(agg + hlp_ref[...])
        s_ref[...] = h
        hp = jnp.concatenate([h[q * 112:(q + 1) * 112] for q in range(8)],
                             axis=1)
        for r in range(R):
            tab_ref[0, r] = jnp.dot(hp, wd_ref[r, 0], preferred_element_type=F32)
            tab_ref[1, r] = jnp.dot(hp, wd_ref[r, 1], preferred_element_type=F32)
        hl_ref[...] = jnp.dot(h, lw_ref[...], preferred_element_type=F32) + b_ref[...]

    return pl.pallas_call(
        body, grid=(NT // Bn,),
        in_specs=[
            pl.BlockSpec((1, Bn // 8, 128), lambda i: (0, i, 0)),
            pl.BlockSpec((1, Bn // 8, 128), lambda i: (1, i, 0)),
            pl.BlockSpec((Bn, 32), lambda i: (i, 0)),
            pl.BlockSpec((R, 2, 256, 128), lambda i: (0, 0, 0, 0)),
            pl.BlockSpec((32, 32), lambda i: (0, 0)),
            pl.BlockSpec((1, 32), lambda i: (0, 0)),
        ],
        out_specs=[
            pl.BlockSpec((Bn, 32), lambda i: (i, 0)),
            pl.BlockSpec((2, R, Bn // 8, 128), lambda i: (0, 0, i, 0)),
            pl.BlockSpec((Bn, 32), lambda i: (i, 0)),
        ],
        out_shape=[
            jax.ShapeDtypeStruct((NT, 32), F32),
            jax.ShapeDtypeStruct((2, R, NT // 8, 128), F32),
            jax.ShapeDtypeStruct((NT, 32), F32),
        ],
    )


@functools.lru_cache(maxsize=None)
def _tc_final_ab():
    Bn = 896

    def body(alo_ref, ahi_ref, hl3_ref, s1_ref, s2_ref, s3_ref, wt_ref, wb_ref,
             ab_ref):
        alo = jnp.concatenate(
            [alo_ref[0][:, q * 16:(q + 1) * 16] for q in range(8)], axis=0)
        ahi = jnp.concatenate(
            [ahi_ref[0][:, q * 16:(q + 1) * 16] for q in range(8)], axis=0)
        agg = jnp.concatenate([alo, ahi], axis=-1)
        s4 = jnp.tanh(agg + hl3_ref[...])
        states = (s1_ref[...], s2_ref[...], s3_ref[...], s4)
        wt = wt_ref[...]
        wb = wb_ref[...]
        a = jnp.dot(states[0], wt[0:32, :], preferred_element_type=F32)
        b = jnp.dot(states[0], wb[0:32, :], preferred_element_type=F32)
        for k in range(1, 4):
            a = a + jnp.dot(states[k], wt[32 * k:32 * k + 32, :],
                            preferred_element_type=F32)
            b = b + jnp.dot(states[k], wb[32 * k:32 * k + 32, :],
                            preferred_element_type=F32)
        ab_ref[0] = a
        ab_ref[1] = b

    return pl.pallas_call(
        body, grid=(NT // Bn,),
        in_specs=[
            pl.BlockSpec((1, Bn // 8, 128), lambda i: (0, i, 0)),
            pl.BlockSpec((1, Bn // 8, 128), lambda i: (1, i, 0)),
            pl.BlockSpec((Bn, 32), lambda i: (i, 0)),
            pl.BlockSpec((Bn, 32), lambda i: (i, 0)),
            pl.BlockSpec((Bn, 32), lambda i: (i, 0)),
            pl.BlockSpec((Bn, 32), lambda i: (i, 0)),
            pl.BlockSpec((128, 128), lambda i: (0, 0)),
            pl.BlockSpec((128, 128), lambda i: (0, 0)),
        ],
        out_specs=[pl.BlockSpec((2, Bn, 128), lambda i: (0, i, 0))],
        out_shape=[jax.ShapeDtypeStruct((2, NT, 128), F32)],
    )


@functools.lru_cache(maxsize=None)
def _tc_out():
    Bn = 896

    def body(su_ref, si_ref, b1_ref, w2_ref, b2_ref, o_ref):
        z = jnp.maximum(su_ref[0] + si_ref[0] + b1_ref[...], 0.0)
        o_ref[...] = jnp.dot(z, w2_ref[...], preferred_element_type=F32) + b2_ref[...]

    return pl.pallas_call(
        body, grid=(NT // Bn,),
        in_specs=[
            pl.BlockSpec((1, Bn, 128), lambda i: (0, i, 0)),
            pl.BlockSpec((1, Bn, 128), lambda i: (1, i, 0)),
            pl.BlockSpec((1, 128), lambda i: (0, 0)),
            pl.BlockSpec((128, 1), lambda i: (0, 0)),
            pl.BlockSpec((1, 1), lambda i: (0, 0)),
        ],
        out_specs=[pl.BlockSpec((Bn, 1), lambda i: (i, 0))],
        out_shape=[jax.ShapeDtypeStruct((NT, 1), F32)],
    )


@functools.lru_cache(maxsize=None)
def _sc_edge():
    @functools.partial(
        pl.kernel, mesh=_mesh(), compiler_params=_SC_PARAMS,
        out_type=jax.ShapeDtypeStruct((2, NT, 16), F32),
        scratch_types=[
            [pltpu.VMEM((CH,), I32)] * NBUF,
            [pltpu.VMEM((CH,), I32)] * NBUF,
            [pltpu.VMEM((CH,), I32)] * NBUF,
            [pltpu.VMEM((CH,), I32)] * NBUF,
            [pltpu.VMEM((CH, 16), F32)] * NBUF,
            pltpu.VMEM((112, 16), F32),
            pltpu.VMEM_SHARED((NT, 16), F32),
            [pltpu.SemaphoreType.DMA] * NBUF,
            [pltpu.SemaphoreType.DMA] * NBUF,
        ],
    )
    def ek(tab_hbm, ei_hbm, out_hbm,
           srcb, dstb, gb, dstg, rows, zb, aggsp, semi, semg):
        c = lax.axis_index("c")
        s = lax.axis_index("s")
        coff = c * (R * NT)
        tbase = s * PW

        def zfill(i, _):
            zb[i] = jnp.zeros((16,), F32)
            return 0
        lax.fori_loop(0, 112, zfill, 0)

        def zcopy(z, _):
            pltpu.sync_copy(zb, aggsp.at[pl.ds(tbase + z * 112, 112)])
            return 0
        lax.fori_loop(0, PW // 112, zcopy, 0)
        plsc.subcore_barrier()

        ebase = s * EPT

        def idx_start(k, b):
            base = ebase + k * CH
            pltpu.async_copy(ei_hbm.at[0, pl.ds(base, CH)], srcb[b], semi[b])
            pltpu.async_copy(ei_hbm.at[1, pl.ds(base, CH)], dstb[b], semi[b])

        def idx_wait(b):
            pltpu.make_async_copy(ei_hbm.at[0, pl.ds(0, CH)], srcb[b], semi[b]).wait()
            pltpu.make_async_copy(ei_hbm.at[0, pl.ds(0, CH)], dstb[b], semi[b]).wait()

        def compute_g(b):
            for j in range(CH // 16):
                sl = pl.ds(j * 16, 16)
                gb[b][sl] = srcb[b][sl] + coff
                dstg[b][sl] = dstb[b][sl]

        def g_start(b):
            pltpu.async_copy(tab_hbm.at[gb[b]], rows[b], semg[b])

        def g_wait_scatter(b):
            pltpu.make_async_copy(tab_hbm.at[gb[b]], rows[b], semg[b]).wait()
            pltpu.sync_copy(rows[b], aggsp.at[dstg[b]], add=True)

        for k0 in range(NBUF):
            idx_start(k0, k0)
        for k0 in range(NBUF - 1):
            idx_wait(k0)
            compute_g(k0)
            g_start(k0)
            idx_start(k0 + NBUF, k0)

        def body(k4, _):
            for j in range(NBUF):
                b = (NBUF - 1 + j) % NBUF
                k = (NBUF - 1) + k4 * NBUF + j
                idx_wait(b)
                compute_g(b)
                g_start(b)
                idx_start(k + NBUF, b)
                g_wait_scatter((b + 1) % NBUF)
            return 0

        lax.fori_loop(0, (NCH - NBUF) // NBUF, body, 0)

        bl = (NCH - 1) % NBUF
        idx_wait(bl)
        compute_g(bl)
        g_start(bl)
        for d in range(NBUF):
            g_wait_scatter((bl + 1 + d) % NBUF)
        for d in range(NBUF - 1):
            idx_wait((bl + 1 + d) % NBUF)

        plsc.subcore_barrier()
        pltpu.sync_copy(aggsp.at[pl.ds(tbase, PW)],
                        out_hbm.at[c, pl.ds(tbase, PW)])

    return ek


@functools.lru_cache(maxsize=None)
def _sc_compact():
    ZW = NSP // 16

    @functools.partial(
        pl.kernel, mesh=_mesh(), compiler_params=_SC_PARAMS,
        out_type=jax.ShapeDtypeStruct((2, NSP), I32),
        scratch_types=[
            pltpu.VMEM((PW,), F32),
            pltpu.VMEM((128,), I32),
            pltpu.VMEM((128,), I32),
            pltpu.VMEM((16,), I32),
            pltpu.VMEM((256,), I32),
            pltpu.VMEM((ZW,), I32),
            pltpu.VMEM_SHARED((NSP,), I32),
            pltpu.VMEM_SHARED((256,), I32),
        ],
    )
    def ck(xu_hbm, xi_hbm, out_hbm,
           condb, posb, valb, cntb, callb, zbc, idxsp, cntsp):
        c = lax.axis_index("c")
        s = lax.axis_index("s")
        base = s * PW
        iota = lax.iota(I32, 16)

        @pl.when(c == 0)
        def _():
            pltpu.sync_copy(xu_hbm.at[pl.ds(base, PW)], condb)

        @pl.when(c == 1)
        def _():
            pltpu.sync_copy(xi_hbm.at[pl.ds(base, PW)], condb)

        def zfill(i, _):
            zbc[pl.ds(i * 16, 16)] = jnp.zeros((16,), I32)
            return 0
        lax.fori_loop(0, ZW // 16, zfill, 0)
        pltpu.sync_copy(zbc, idxsp.at[pl.ds(s * ZW, ZW)])

        one16 = jnp.ones((16,), I32)
        zero16 = jnp.zeros((16,), I32)

        def cnt(i, acc):
            f = condb[pl.ds(i * 16, 16)]
            return acc + jnp.sum(jnp.where(f != 0.0, one16, zero16))
        total = lax.fori_loop(0, PW // 16, cnt, jnp.zeros((), I32))
        cntb[...] = total + jnp.zeros((16,), I32)
        pltpu.sync_copy(cntb, cntsp.at[pl.ds(s * 16, 16)])
        plsc.subcore_barrier()

        pltpu.sync_copy(cntsp, callb)
        cvec = plsc.load_gather(callb, [iota * 16])
        excl0 = jnp.sum(jnp.where(iota < s, cvec, jnp.zeros((16,), I32)))

        def body(ko, off):
            for j in range(8):
                i = ko * 8 + j
                f = condb[pl.ds(i * 16, 16)]
                v = jnp.where(f != 0.0, one16, zero16)
                incl = plsc.cumsum(v)
                pos = off + incl - 1
                posm = jnp.where(v == 1, pos, NT + iota)
                sl = pl.ds(j * 16, 16)
                posb[sl] = posm
                valb[sl] = base + i * 16 + iota
                off = off + jnp.sum(v)
            pltpu.sync_copy(valb, idxsp.at[posb])
            return off

        lax.fori_loop(0, PW // 128, body, excl0)
        plsc.subcore_barrier()
        pltpu.sync_copy(idxsp.at[pl.ds(base, PW)], out_hbm.at[c, pl.ds(base, PW)])

        @pl.when(s == 15)
        def _():
            pltpu.sync_copy(idxsp.at[pl.ds(NT, 512)],
                            out_hbm.at[c, pl.ds(NT, 512)])

    return ck


@functools.lru_cache(maxsize=None)
def _sc_gather_ab():
    GC = 112
    NCHG = PW // GC
    NBG = 4

    @functools.partial(
        pl.kernel, mesh=_mesh(), compiler_params=_SC_PARAMS,
        out_type=jax.ShapeDtypeStruct((2, NT, 128), F32),
        scratch_types=[
            [pltpu.VMEM((GC,), I32)] * NBG,
            [pltpu.VMEM((GC,), I32)] * NBG,
            [pltpu.VMEM((GC, 128), F32)] * NBG,
            [pltpu.SemaphoreType.DMA] * NBG,
            [pltpu.SemaphoreType.DMA] * NBG,
        ],
    )
    def gk(uij_hbm, ab_hbm, out_hbm, idxb, gb, rows, semi, semg):
        c = lax.axis_index("c")
        s = lax.axis_index("s")
        base = s * PW
        coff = c * NT

        def idx_start(k, b):
            pltpu.async_copy(uij_hbm.at[c, pl.ds(base + k * GC, GC)],
                             idxb[b], semi[b])

        def idx_wait(b):
            pltpu.make_async_copy(uij_hbm.at[c, pl.ds(base, GC)],
                                  idxb[b], semi[b]).wait()

        def compute_g(b):
            for j in range(GC // 16):
                sl = pl.ds(j * 16, 16)
                gb[b][sl] = idxb[b][sl] + coff

        def g_start(b):
            pltpu.async_copy(ab_hbm.at[gb[b]], rows[b], semg[b])

        def g_wait_store(b, k):
            pltpu.make_async_copy(ab_hbm.at[gb[b]], rows[b], semg[b]).wait()
            pltpu.sync_copy(rows[b], out_hbm.at[c, pl.ds(base + k * GC, GC)])

        for k0 in range(NBG):
            idx_start(k0, k0)
        for k0 in range(NBG - 1):
            idx_wait(k0)
            compute_g(k0)
            g_start(k0)
            idx_start(k0 + NBG, k0)

        def body(k4, _):
            for j in range(NBG):
                b = (NBG - 1 + j) % NBG
                k = (NBG - 1) + k4 * NBG + j
                idx_wait(b)
                compute_g(b)
                g_start(b)
                idx_start(k + NBG, b)
                g_wait_store((b + 1) % NBG, k - NBG + 1)
            return 0

        lax.fori_loop(0, (NCHG - NBG) // NBG, body, 0)

        bl = (NCHG - 1) % NBG
        idx_wait(bl)
        compute_g(bl)
        g_start(bl)
        for d in range(NBG):
            bo = (bl + 1 + d) % NBG
            g_wait_store(bo, NCHG - NBG + d)
        for d in range(NBG - 1):
            idx_wait((bl + 1 + d) % NBG)

    return gk


def _wdiag(V, comp):
    din = V.shape[1]
    W3 = jnp.dot(comp, V.reshape(NB, din * 32)).reshape(R, din, 32)
    rows = []
    for r in range(R):
        rows.append(jnp.stack([
            block_diag(*([W3[r, :, h * 16:(h + 1) * 16]] * 8)) for h in (0, 1)
        ]))
    return jnp.stack(rows)


def kernel(x, edge_index, etype, V0, comp0, loop0, b0, V1, comp1, loop1, b1,
           V2, comp2, loop2, b2, V3, comp3, loop3, b3,
           lin1_w, lin1_b, lin2_w, lin2_b):
    npad = EPAD + NBUF * CH - E
    pad_src = jnp.zeros((1, npad), I32)
    pad_dst = jnp.full((1, npad), N, I32)
    eip = jnp.concatenate([edge_index, jnp.concatenate([pad_src, pad_dst], 0)], 1)
    etp = jnp.pad(etype, (0, npad))
    d1 = eip // 112
    rem = eip - d1 * 112
    off = jnp.stack([etp * NT, jnp.zeros_like(etp)])
    eip = (d1 >> 3) * 896 + rem * 8 + (d1 & 7) + off
    xu = jnp.pad(x[:, 0], (0, NT - N))
    xi = jnp.pad(x[:, 1], (0, NT - N))
    xp = jnp.pad(x, ((0, NT - N), (0, 0)))

    l0 = _tc_layer0()
    ll = _tc_layer()
    ek = _sc_edge()
    tab0, hl0 = l0(xp, _wdiag(V0, comp0), loop0, b0.reshape(1, 32))
    agg0 = ek(tab0.reshape(2 * R * NT, 16), eip).reshape(2, NT // 8, 128)
    s1, tab1, hl1 = ll(agg0, agg0, hl0, _wdiag(V1, comp1), loop1, b1.reshape(1, 32))
    agg1 = ek(tab1.reshape(2 * R * NT, 16), eip).reshape(2, NT // 8, 128)
    s2, tab2, hl2 = ll(agg1, agg1, hl1, _wdiag(V2, comp2), loop2, b2.reshape(1, 32))
    agg2 = ek(tab2.reshape(2 * R * NT, 16), eip).reshape(2, NT // 8, 128)
    s3, tab3, hl3 = ll(agg2, agg2, hl2, _wdiag(V3, comp3), loop3, b3.reshape(1, 32))
    agg3 = ek(tab3.reshape(2 * R * NT, 16), eip).reshape(2, NT // 8, 128)

    ab, = _tc_final_ab()(agg3, agg3, hl3, s1, s2, s3,
                         lin1_w[:128, :], lin1_w[128:, :])
    uij = _sc_compact()(xu, xi)
    s2g = _sc_gather_ab()(uij, ab.reshape(2 * NT, 128))
    o, = _tc_out()(s2g, s2g, lin1_b.reshape(1, 128), lin2_w, lin2_b.reshape(1, 1))
    return o[:N, 0]


# speedup vs baseline: 1.1808x; 1.1171x over previous
import functools

import jax
import jax.numpy as jnp
from jax import lax
from jax.scipy.linalg import block_diag
from jax.experimental import pallas as pl
from jax.experimental.pallas import tpu as pltpu
from jax.experimental.pallas import tpu_sc as plsc

N = 100000
E = 1600000
R = 5
NB = 2
NT = 100352
PW = 6272
CH = 128
NCH = 784
EPT = NCH * CH
EPAD = 16 * EPT
NBUF = 8
NSP = NT + 512
F32 = jnp.float32
I32 = jnp.int32


def _mesh():
    return plsc.VectorSubcoreMesh(core_axis_name="c", subcore_axis_name="s")


_SC_PARAMS = pltpu.CompilerParams(use_tc_tiling_on_sc=False,
                                  needs_layout_passes=False)


def _pack_rows(h, Bn):
    return jnp.concatenate([
        jnp.concatenate([h[g * 896 + q * 112: g * 896 + (q + 1) * 112]
                         for q in range(8)], axis=1)
        for g in range(Bn // 896)], axis=0)


def _unpack_cols(a, Bn):
    return jnp.concatenate([
        jnp.concatenate([a[g * 112:(g + 1) * 112, q * 16:(q + 1) * 16]
                         for q in range(8)], axis=0)
        for g in range(Bn // 896)], axis=0)


@functools.lru_cache(maxsize=None)
def _tc_layer0():
    Bn = 1792

    def body(x_ref, wd_ref, lw_ref, b_ref, tab_ref, hl_ref):
        h = x_ref[...]
        hp = _pack_rows(h, Bn)
        for r in range(R):
            tab_ref[0, r] = jnp.dot(hp, wd_ref[r, 0], preferred_element_type=F32)
            tab_ref[1, r] = jnp.dot(hp, wd_ref[r, 1], preferred_element_type=F32)
        hl_ref[...] = jnp.dot(h, lw_ref[...], preferred_element_type=F32) + b_ref[...]

    return pl.pallas_call(
        body, grid=(NT // Bn,),
        in_specs=[
            pl.BlockSpec((Bn, 4), lambda i: (i, 0)),
            pl.BlockSpec((R, 2, 32, 128), lambda i: (0, 0, 0, 0)),
            pl.BlockSpec((4, 32), lambda i: (0, 0)),
            pl.BlockSpec((1, 32), lambda i: (0, 0)),
        ],
        out_specs=[
            pl.BlockSpec((2, R, Bn // 8, 128), lambda i: (0, 0, i, 0)),
            pl.BlockSpec((Bn, 32), lambda i: (i, 0)),
        ],
        out_shape=[
            jax.ShapeDtypeStruct((2, R, NT // 8, 128), F32),
            jax.ShapeDtypeStruct((NT, 32), F32),
        ],
    )


@functools.lru_cache(maxsize=None)
def _tc_layer():
    Bn = 1792

    def body(alo_ref, ahi_ref, hlp_ref, wd_ref, lw_ref, b_ref,
             s_ref, tab_ref, hl_ref):
        alo = _unpack_cols(alo_ref[0], Bn)
        ahi = _unpack_cols(ahi_ref[0], Bn)
        agg = jnp.concatenate([alo, ahi], axis=-1)
        h = jnp.tanh(agg + hlp_ref[...])
        s_ref[...] = h
        hp = _pack_rows(h, Bn)
        for r in range(R):
            tab_ref[0, r] = jnp.dot(hp, wd_ref[r, 0], preferred_element_type=F32)
            tab_ref[1, r] = jnp.dot(hp, wd_ref[r, 1], preferred_element_type=F32)
        hl_ref[...] = jnp.dot(h, lw_ref[...], preferred_element_type=F32) + b_ref[...]

    return pl.pallas_call(
        body, grid=(NT // Bn,),
        in_specs=[
            pl.BlockSpec((1, Bn // 8, 128), lambda i: (0, i, 0)),
            pl.BlockSpec((1, Bn // 8, 128), lambda i: (1, i, 0)),
            pl.BlockSpec((Bn, 32), lambda i: (i, 0)),
            pl.BlockSpec((R, 2, 256, 128), lambda i: (0, 0, 0, 0)),
            pl.BlockSpec((32, 32), lambda i: (0, 0)),
            pl.BlockSpec((1, 32), lambda i: (0, 0)),
        ],
        out_specs=[
            pl.BlockSpec((Bn, 32), lambda i: (i, 0)),
            pl.BlockSpec((2, R, Bn // 8, 128), lambda i: (0, 0, i, 0)),
            pl.BlockSpec((Bn, 32), lambda i: (i, 0)),
        ],
        out_shape=[
            jax.ShapeDtypeStruct((NT, 32), F32),
            jax.ShapeDtypeStruct((2, R, NT // 8, 128), F32),
            jax.ShapeDtypeStruct((NT, 32), F32),
        ],
    )


@functools.lru_cache(maxsize=None)
def _tc_final_ab():
    Bn = 1792

    def body(alo_ref, ahi_ref, hl3_ref, s1_ref, s2_ref, s3_ref, wt_ref, wb_ref,
             ab_ref):
        alo = _unpack_cols(alo_ref[0], Bn)
        ahi = _unpack_cols(ahi_ref[0], Bn)
        agg = jnp.concatenate([alo, ahi], axis=-1)
        s4 = jnp.tanh(agg + hl3_ref[...])
        states = (s1_ref[...], s2_ref[...], s3_ref[...], s4)
        wt = wt_ref[...]
        wb = wb_ref[...]
        a = jnp.dot(states[0], wt[0:32, :], preferred_element_type=F32)
        b = jnp.dot(states[0], wb[0:32, :], preferred_element_type=F32)
        for k in range(1, 4):
            a = a + jnp.dot(states[k], wt[32 * k:32 * k + 32, :],
                            preferred_element_type=F32)
            b = b + jnp.dot(states[k], wb[32 * k:32 * k + 32, :],
                            preferred_element_type=F32)
        ab_ref[0] = a
        ab_ref[1] = b

    return pl.pallas_call(
        body, grid=(NT // Bn,),
        in_specs=[
            pl.BlockSpec((1, Bn // 8, 128), lambda i: (0, i, 0)),
            pl.BlockSpec((1, Bn // 8, 128), lambda i: (1, i, 0)),
            pl.BlockSpec((Bn, 32), lambda i: (i, 0)),
            pl.BlockSpec((Bn, 32), lambda i: (i, 0)),
            pl.BlockSpec((Bn, 32), lambda i: (i, 0)),
            pl.BlockSpec((Bn, 32), lambda i: (i, 0)),
            pl.BlockSpec((128, 128), lambda i: (0, 0)),
            pl.BlockSpec((128, 128), lambda i: (0, 0)),
        ],
        out_specs=[pl.BlockSpec((2, Bn, 128), lambda i: (0, i, 0))],
        out_shape=[jax.ShapeDtypeStruct((2, NT, 128), F32)],
    )


@functools.lru_cache(maxsize=None)
def _tc_out():
    Bn = 1792

    def body(su_ref, si_ref, b1_ref, w2_ref, b2_ref, o_ref):
        z = jnp.maximum(su_ref[0] + si_ref[0] + b1_ref[...], 0.0)
        o_ref[...] = jnp.dot(z, w2_ref[...], preferred_element_type=F32) + b2_ref[...]

    return pl.pallas_call(
        body, grid=(NT // Bn,),
        in_specs=[
            pl.BlockSpec((1, Bn, 128), lambda i: (0, i, 0)),
            pl.BlockSpec((1, Bn, 128), lambda i: (1, i, 0)),
            pl.BlockSpec((1, 128), lambda i: (0, 0)),
            pl.BlockSpec((128, 1), lambda i: (0, 0)),
            pl.BlockSpec((1, 1), lambda i: (0, 0)),
        ],
        out_specs=[pl.BlockSpec((Bn, 1), lambda i: (i, 0))],
        out_shape=[jax.ShapeDtypeStruct((NT, 1), F32)],
    )


@functools.lru_cache(maxsize=None)
def _sc_edge():
    @functools.partial(
        pl.kernel, mesh=_mesh(), compiler_params=_SC_PARAMS,
        out_type=jax.ShapeDtypeStruct((2, NT, 16), F32),
        scratch_types=[
            [pltpu.VMEM((CH,), I32)] * NBUF,
            [pltpu.VMEM((CH,), I32)] * NBUF,
            [pltpu.VMEM((CH,), I32)] * NBUF,
            [pltpu.VMEM((CH,), I32)] * NBUF,
            [pltpu.VMEM((CH, 16), F32)] * NBUF,
            pltpu.VMEM((112, 16), F32),
            pltpu.VMEM_SHARED((NT, 16), F32),
            [pltpu.SemaphoreType.DMA] * NBUF,
            [pltpu.SemaphoreType.DMA] * NBUF,
        ],
    )
    def ek(tab_hbm, ei_hbm, out_hbm,
           srcb, dstb, gb, dstg, rows, zb, aggsp, semi, semg):
        c = lax.axis_index("c")
        s = lax.axis_index("s")
        coff = c * (R * NT)
        tbase = s * PW

        def zfill(i, _):
            zb[i] = jnp.zeros((16,), F32)
            return 0
        lax.fori_loop(0, 112, zfill, 0)

        def zcopy(z, _):
            pltpu.sync_copy(zb, aggsp.at[pl.ds(tbase + z * 112, 112)])
            return 0
        lax.fori_loop(0, PW // 112, zcopy, 0)
        plsc.subcore_barrier()

        ebase = s * EPT

        def idx_start(k, b):
            base = ebase + k * CH
            pltpu.async_copy(ei_hbm.at[0, pl.ds(base, CH)], srcb[b], semi[b])
            pltpu.async_copy(ei_hbm.at[1, pl.ds(base, CH)], dstb[b], semi[b])

        def idx_wait(b):
            pltpu.make_async_copy(ei_hbm.at[0, pl.ds(0, CH)], srcb[b], semi[b]).wait()
            pltpu.make_async_copy(ei_hbm.at[0, pl.ds(0, CH)], dstb[b], semi[b]).wait()

        def compute_g(b):
            for j in range(CH // 16):
                sl = pl.ds(j * 16, 16)
                gb[b][sl] = srcb[b][sl] + coff
                dstg[b][sl] = dstb[b][sl]

        def g_start(b):
            pltpu.async_copy(tab_hbm.at[gb[b]], rows[b], semg[b])

        def g_wait_scatter(b):
            pltpu.make_async_copy(tab_hbm.at[gb[b]], rows[b], semg[b]).wait()
            pltpu.sync_copy(rows[b], aggsp.at[dstg[b]], add=True)

        for k0 in range(NBUF):
            idx_start(k0, k0)
        for k0 in range(NBUF - 1):
            idx_wait(k0)
            compute_g(k0)
            g_start(k0)
            idx_start(k0 + NBUF, k0)

        def body(k4, _):
            for j in range(NBUF):
                b = (NBUF - 1 + j) % NBUF
                k = (NBUF - 1) + k4 * NBUF + j
                idx_wait(b)
                compute_g(b)
                g_start(b)
                idx_start(k + NBUF, b)
                g_wait_scatter((b + 1) % NBUF)
            return 0

        lax.fori_loop(0, (NCH - NBUF) // NBUF, body, 0)

        bl = (NCH - 1) % NBUF
        idx_wait(bl)
        compute_g(bl)
        g_start(bl)
        for d in range(NBUF):
            g_wait_scatter((bl + 1 + d) % NBUF)
        for d in range(NBUF - 1):
            idx_wait((bl + 1 + d) % NBUF)

        plsc.subcore_barrier()
        pltpu.sync_copy(aggsp.at[pl.ds(tbase, PW)],
                        out_hbm.at[c, pl.ds(tbase, PW)])

    return ek


@functools.lru_cache(maxsize=None)
def _sc_compact():
    ZW = NSP // 16

    @functools.partial(
        pl.kernel, mesh=_mesh(), compiler_params=_SC_PARAMS,
        out_type=jax.ShapeDtypeStruct((2, NSP), I32),
        scratch_types=[
            pltpu.VMEM((PW,), F32),
            pltpu.VMEM((128,), I32),
            pltpu.VMEM((128,), I32),
            pltpu.VMEM((16,), I32),
            pltpu.VMEM((256,), I32),
            pltpu.VMEM((ZW,), I32),
            pltpu.VMEM_SHARED((NSP,), I32),
            pltpu.VMEM_SHARED((256,), I32),
        ],
    )
    def ck(xu_hbm, xi_hbm, out_hbm,
           condb, posb, valb, cntb, callb, zbc, idxsp, cntsp):
        c = lax.axis_index("c")
        s = lax.axis_index("s")
        base = s * PW
        iota = lax.iota(I32, 16)

        @pl.when(c == 0)
        def _():
            pltpu.sync_copy(xu_hbm.at[pl.ds(base, PW)], condb)

        @pl.when(c == 1)
        def _():
            pltpu.sync_copy(xi_hbm.at[pl.ds(base, PW)], condb)

        def zfill(i, _):
            zbc[pl.ds(i * 16, 16)] = jnp.zeros((16,), I32)
            return 0
        lax.fori_loop(0, ZW // 16, zfill, 0)
        pltpu.sync_copy(zbc, idxsp.at[pl.ds(s * ZW, ZW)])

        one16 = jnp.ones((16,), I32)
        zero16 = jnp.zeros((16,), I32)

        def cnt(i, acc):
            f = condb[pl.ds(i * 16, 16)]
            return acc + jnp.sum(jnp.where(f != 0.0, one16, zero16))
        total = lax.fori_loop(0, PW // 16, cnt, jnp.zeros((), I32))
        cntb[...] = total + jnp.zeros((16,), I32)
        pltpu.sync_copy(cntb, cntsp.at[pl.ds(s * 16, 16)])
        plsc.subcore_barrier()

        pltpu.sync_copy(cntsp, callb)
        cvec = plsc.load_gather(callb, [iota * 16])
        excl0 = jnp.sum(jnp.where(iota < s, cvec, jnp.zeros((16,), I32)))

        def body(ko, off):
            for j in range(8):
                i = ko * 8 + j
                f = condb[pl.ds(i * 16, 16)]
                v = jnp.where(f != 0.0, one16, zero16)
                incl = plsc.cumsum(v)
                pos = off + incl - 1
                posm = jnp.where(v == 1, pos, NT + iota)
                sl = pl.ds(j * 16, 16)
                posb[sl] = posm
                valb[sl] = base + i * 16 + iota
                off = off + jnp.sum(v)
            pltpu.sync_copy(valb, idxsp.at[posb])
            return off

        lax.fori_loop(0, PW // 128, body, excl0)
        plsc.subcore_barrier()
        pltpu.sync_copy(idxsp.at[pl.ds(base, PW)], out_hbm.at[c, pl.ds(base, PW)])

        @pl.when(s == 15)
        def _():
            pltpu.sync_copy(idxsp.at[pl.ds(NT, 512)],
                            out_hbm.at[c, pl.ds(NT, 512)])

    return ck


@functools.lru_cache(maxsize=None)
def _sc_gather_ab():
    GC = 112
    NCHG = PW // GC
    NBG = 4

    @functools.partial(
        pl.kernel, mesh=_mesh(), compiler_params=_SC_PARAMS,
        out_type=jax.ShapeDtypeStruct((2, NT, 128), F32),
        scratch_types=[
            [pltpu.VMEM((GC,), I32)] * NBG,
            [pltpu.VMEM((GC,), I32)] * NBG,
            [pltpu.VMEM((GC, 128), F32)] * NBG,
            [pltpu.SemaphoreType.DMA] * NBG,
            [pltpu.SemaphoreType.DMA] * NBG,
        ],
    )
    def gk(uij_hbm, ab_hbm, out_hbm, idxb, gb, rows, semi, semg):
        c = lax.axis_index("c")
        s = lax.axis_index("s")
        base = s * PW
        coff = c * NT

        def idx_start(k, b):
            pltpu.async_copy(uij_hbm.at[c, pl.ds(base + k * GC, GC)],
                             idxb[b], semi[b])

        def idx_wait(b):
            pltpu.make_async_copy(uij_hbm.at[c, pl.ds(base, GC)],
                                  idxb[b], semi[b]).wait()

        def compute_g(b):
            for j in range(GC // 16):
                sl = pl.ds(j * 16, 16)
                gb[b][sl] = idxb[b][sl] + coff

        def g_start(b):
            pltpu.async_copy(ab_hbm.at[gb[b]], rows[b], semg[b])

        def g_wait_store(b, k):
            pltpu.make_async_copy(ab_hbm.at[gb[b]], rows[b], semg[b]).wait()
            pltpu.sync_copy(rows[b], out_hbm.at[c, pl.ds(base + k * GC, GC)])

        for k0 in range(NBG):
            idx_start(k0, k0)
        for k0 in range(NBG - 1):
            idx_wait(k0)
            compute_g(k0)
            g_start(k0)
            idx_start(k0 + NBG, k0)

        def body(k4, _):
            for j in range(NBG):
                b = (NBG - 1 + j) % NBG
                k = (NBG - 1) + k4 * NBG + j
                idx_wait(b)
                compute_g(b)
                g_start(b)
                idx_start(k + NBG, b)
                g_wait_store((b + 1) % NBG, k - NBG + 1)
            return 0

        lax.fori_loop(0, (NCHG - NBG) // NBG, body, 0)

        bl = (NCHG - 1) % NBG
        idx_wait(bl)
        compute_g(bl)
        g_start(bl)
        for d in range(NBG):
            bo = (bl + 1 + d) % NBG
            g_wait_store(bo, NCHG - NBG + d)
        for d in range(NBG - 1):
            idx_wait((bl + 1 + d) % NBG)

    return gk


def _wdiag(V, comp):
    din = V.shape[1]
    W3 = jnp.dot(comp, V.reshape(NB, din * 32)).reshape(R, din, 32)
    rows = []
    for r in range(R):
        rows.append(jnp.stack([
            block_diag(*([W3[r, :, h * 16:(h + 1) * 16]] * 8)) for h in (0, 1)
        ]))
    return jnp.stack(rows)


def kernel(x, edge_index, etype, V0, comp0, loop0, b0, V1, comp1, loop1, b1,
           V2, comp2, loop2, b2, V3, comp3, loop3, b3,
           lin1_w, lin1_b, lin2_w, lin2_b):
    npad = EPAD + NBUF * CH - E
    pad_src = jnp.zeros((1, npad), I32)
    pad_dst = jnp.full((1, npad), N, I32)
    eip = jnp.concatenate([edge_index, jnp.concatenate([pad_src, pad_dst], 0)], 1)
    etp = jnp.pad(etype, (0, npad))
    d1 = eip // 112
    rem = eip - d1 * 112
    off = jnp.stack([etp * NT, jnp.zeros_like(etp)])
    eip = (d1 >> 3) * 896 + rem * 8 + (d1 & 7) + off
    xu = jnp.pad(x[:, 0], (0, NT - N))
    xi = jnp.pad(x[:, 1], (0, NT - N))
    xp = jnp.pad(x, ((0, NT - N), (0, 0)))

    l0 = _tc_layer0()
    ll = _tc_layer()
    ek = _sc_edge()
    tab0, hl0 = l0(xp, _wdiag(V0, comp0), loop0, b0.reshape(1, 32))
    agg0 = ek(tab0.reshape(2 * R * NT, 16), eip).reshape(2, NT // 8, 128)
    s1, tab1, hl1 = ll(agg0, agg0, hl0, _wdiag(V1, comp1), loop1, b1.reshape(1, 32))
    agg1 = ek(tab1.reshape(2 * R * NT, 16), eip).reshape(2, NT // 8, 128)
    s2, tab2, hl2 = ll(agg1, agg1, hl1, _wdiag(V2, comp2), loop2, b2.reshape(1, 32))
    agg2 = ek(tab2.reshape(2 * R * NT, 16), eip).reshape(2, NT // 8, 128)
    s3, tab3, hl3 = ll(agg2, agg2, hl2, _wdiag(V3, comp3), loop3, b3.reshape(1, 32))
    agg3 = ek(tab3.reshape(2 * R * NT, 16), eip).reshape(2, NT // 8, 128)

    ab, = _tc_final_ab()(agg3, agg3, hl3, s1, s2, s3,
                         lin1_w[:128, :], lin1_w[128:, :])
    uij = _sc_compact()(xu, xi)
    s2g = _sc_gather_ab()(uij, ab.reshape(2 * NT, 128))
    o, = _tc_out()(s2g, s2g, lin1_b.reshape(1, 128), lin2_w, lin2_b.reshape(1, 1))
    return o[:N, 0]


# speedup vs baseline: 1.2584x; 1.0658x over previous
import functools

import jax
import jax.numpy as jnp
from jax import lax
from jax.scipy.linalg import block_diag
from jax.experimental import pallas as pl
from jax.experimental.pallas import tpu as pltpu
from jax.experimental.pallas import tpu_sc as plsc

N = 100000
E = 1600000
R = 5
NB = 2
NT = 100352
PW = 6272
CH = 128
NCH = 784
EPT = NCH * CH
EPAD = 16 * EPT
NBUF = 8
NSP = NT + 512
F32 = jnp.float32
I32 = jnp.int32


def _mesh():
    return plsc.VectorSubcoreMesh(core_axis_name="c", subcore_axis_name="s")


_SC_PARAMS = pltpu.CompilerParams(use_tc_tiling_on_sc=False,
                                  needs_layout_passes=False)


def _pack_rows(h, Bn):
    return jnp.concatenate([
        jnp.concatenate([h[g * 896 + q * 112: g * 896 + (q + 1) * 112]
                         for q in range(8)], axis=1)
        for g in range(Bn // 896)], axis=0)


def _unpack_cols(a, Bn):
    return jnp.concatenate([
        jnp.concatenate([a[g * 112:(g + 1) * 112, q * 16:(q + 1) * 16]
                         for q in range(8)], axis=0)
        for g in range(Bn // 896)], axis=0)


@functools.lru_cache(maxsize=None)
def _tc_layer0():
    Bn = 3584

    def body(x_ref, wd_ref, lw_ref, b_ref, tab_ref, hl_ref):
        h = x_ref[...]
        hp = _pack_rows(h, Bn)
        for r in range(R):
            tab_ref[0, r] = jnp.dot(hp, wd_ref[r, 0], preferred_element_type=F32)
            tab_ref[1, r] = jnp.dot(hp, wd_ref[r, 1], preferred_element_type=F32)
        hl_ref[...] = jnp.dot(h, lw_ref[...], preferred_element_type=F32) + b_ref[...]

    return pl.pallas_call(
        body, grid=(NT // Bn,),
        in_specs=[
            pl.BlockSpec((Bn, 4), lambda i: (i, 0)),
            pl.BlockSpec((R, 2, 32, 128), lambda i: (0, 0, 0, 0)),
            pl.BlockSpec((4, 32), lambda i: (0, 0)),
            pl.BlockSpec((1, 32), lambda i: (0, 0)),
        ],
        out_specs=[
            pl.BlockSpec((2, R, Bn // 8, 128), lambda i: (0, 0, i, 0)),
            pl.BlockSpec((Bn, 32), lambda i: (i, 0)),
        ],
        out_shape=[
            jax.ShapeDtypeStruct((2, R, NT // 8, 128), F32),
            jax.ShapeDtypeStruct((NT, 32), F32),
        ],
    )


@functools.lru_cache(maxsize=None)
def _tc_layer():
    Bn = 3584

    def body(alo_ref, ahi_ref, hlp_ref, wd_ref, lw_ref, b_ref,
             s_ref, tab_ref, hl_ref):
        alo = _unpack_cols(alo_ref[0], Bn)
        ahi = _unpack_cols(ahi_ref[0], Bn)
        agg = jnp.concatenate([alo, ahi], axis=-1)
        h = jnp.tanh(agg + hlp_ref[...])
        s_ref[...] = h
        hp = _pack_rows(h, Bn)
        for r in range(R):
            tab_ref[0, r] = jnp.dot(hp, wd_ref[r, 0], preferred_element_type=F32)
            tab_ref[1, r] = jnp.dot(hp, wd_ref[r, 1], preferred_element_type=F32)
        hl_ref[...] = jnp.dot(h, lw_ref[...], preferred_element_type=F32) + b_ref[...]

    return pl.pallas_call(
        body, grid=(NT // Bn,),
        in_specs=[
            pl.BlockSpec((1, Bn // 8, 128), lambda i: (0, i, 0)),
            pl.BlockSpec((1, Bn // 8, 128), lambda i: (1, i, 0)),
            pl.BlockSpec((Bn, 32), lambda i: (i, 0)),
            pl.BlockSpec((R, 2, 256, 128), lambda i: (0, 0, 0, 0)),
            pl.BlockSpec((32, 32), lambda i: (0, 0)),
            pl.BlockSpec((1, 32), lambda i: (0, 0)),
        ],
        out_specs=[
            pl.BlockSpec((Bn, 32), lambda i: (i, 0)),
            pl.BlockSpec((2, R, Bn // 8, 128), lambda i: (0, 0, i, 0)),
            pl.BlockSpec((Bn, 32), lambda i: (i, 0)),
        ],
        out_shape=[
            jax.ShapeDtypeStruct((NT, 32), F32),
            jax.ShapeDtypeStruct((2, R, NT // 8, 128), F32),
            jax.ShapeDtypeStruct((NT, 32), F32),
        ],
    )


@functools.lru_cache(maxsize=None)
def _tc_final_ab():
    Bn = 3584

    def body(alo_ref, ahi_ref, hl3_ref, s1_ref, s2_ref, s3_ref, wt_ref, wb_ref,
             ab_ref):
        alo = _unpack_cols(alo_ref[0], Bn)
        ahi = _unpack_cols(ahi_ref[0], Bn)
        agg = jnp.concatenate([alo, ahi], axis=-1)
        s4 = jnp.tanh(agg + hl3_ref[...])
        states = (s1_ref[...], s2_ref[...], s3_ref[...], s4)
        wt = wt_ref[...]
        wb = wb_ref[...]
        a = jnp.dot(states[0], wt[0:32, :], preferred_element_type=F32)
        b = jnp.dot(states[0], wb[0:32, :], preferred_element_type=F32)
        for k in range(1, 4):
            a = a + jnp.dot(states[k], wt[32 * k:32 * k + 32, :],
                            preferred_element_type=F32)
            b = b + jnp.dot(states[k], wb[32 * k:32 * k + 32, :],
                            preferred_element_type=F32)
        ab_ref[0] = a
        ab_ref[1] = b

    return pl.pallas_call(
        body, grid=(NT // Bn,),
        in_specs=[
            pl.BlockSpec((1, Bn // 8, 128), lambda i: (0, i, 0)),
            pl.BlockSpec((1, Bn // 8, 128), lambda i: (1, i, 0)),
            pl.BlockSpec((Bn, 32), lambda i: (i, 0)),
            pl.BlockSpec((Bn, 32), lambda i: (i, 0)),
            pl.BlockSpec((Bn, 32), lambda i: (i, 0)),
            pl.BlockSpec((Bn, 32), lambda i: (i, 0)),
            pl.BlockSpec((128, 128), lambda i: (0, 0)),
            pl.BlockSpec((128, 128), lambda i: (0, 0)),
        ],
        out_specs=[pl.BlockSpec((2, Bn, 128), lambda i: (0, i, 0))],
        out_shape=[jax.ShapeDtypeStruct((2, NT, 128), F32)],
    )


@functools.lru_cache(maxsize=None)
def _tc_out():
    Bn = 3584

    def body(su_ref, si_ref, b1_ref, w2_ref, b2_ref, o_ref):
        z = jnp.maximum(su_ref[0] + si_ref[0] + b1_ref[...], 0.0)
        o_ref[...] = jnp.dot(z, w2_ref[...], preferred_element_type=F32) + b2_ref[...]

    return pl.pallas_call(
        body, grid=(NT // Bn,),
        in_specs=[
            pl.BlockSpec((1, Bn, 128), lambda i: (0, i, 0)),
            pl.BlockSpec((1, Bn, 128), lambda i: (1, i, 0)),
            pl.BlockSpec((1, 128), lambda i: (0, 0)),
            pl.BlockSpec((128, 1), lambda i: (0, 0)),
            pl.BlockSpec((1, 1), lambda i: (0, 0)),
        ],
        out_specs=[pl.BlockSpec((Bn, 1), lambda i: (i, 0))],
        out_shape=[jax.ShapeDtypeStruct((NT, 1), F32)],
    )


@functools.lru_cache(maxsize=None)
def _sc_edge():
    @functools.partial(
        pl.kernel, mesh=_mesh(), compiler_params=_SC_PARAMS,
        out_type=jax.ShapeDtypeStruct((2, NT, 16), F32),
        scratch_types=[
            [pltpu.VMEM((CH,), I32)] * NBUF,
            [pltpu.VMEM((CH,), I32)] * NBUF,
            [pltpu.VMEM((CH,), I32)] * NBUF,
            [pltpu.VMEM((CH,), I32)] * NBUF,
            [pltpu.VMEM((CH, 16), F32)] * NBUF,
            pltpu.VMEM((112, 16), F32),
            pltpu.VMEM_SHARED((NT, 16), F32),
            [pltpu.SemaphoreType.DMA] * NBUF,
            [pltpu.SemaphoreType.DMA] * NBUF,
        ],
    )
    def ek(tab_hbm, ei_hbm, out_hbm,
           srcb, dstb, gb, dstg, rows, zb, aggsp, semi, semg):
        c = lax.axis_index("c")
        s = lax.axis_index("s")
        coff = c * (R * NT)
        tbase = s * PW

        def zfill(i, _):
            zb[i] = jnp.zeros((16,), F32)
            return 0
        lax.fori_loop(0, 112, zfill, 0)

        def zcopy(z, _):
            pltpu.sync_copy(zb, aggsp.at[pl.ds(tbase + z * 112, 112)])
            return 0
        lax.fori_loop(0, PW // 112, zcopy, 0)
        plsc.subcore_barrier()

        ebase = s * EPT

        def idx_start(k, b):
            base = ebase + k * CH
            pltpu.async_copy(ei_hbm.at[0, pl.ds(base, CH)], srcb[b], semi[b])
            pltpu.async_copy(ei_hbm.at[1, pl.ds(base, CH)], dstb[b], semi[b])

        def idx_wait(b):
            pltpu.make_async_copy(ei_hbm.at[0, pl.ds(0, CH)], srcb[b], semi[b]).wait()
            pltpu.make_async_copy(ei_hbm.at[0, pl.ds(0, CH)], dstb[b], semi[b]).wait()

        def compute_g(b):
            for j in range(CH // 16):
                sl = pl.ds(j * 16, 16)
                gb[b][sl] = srcb[b][sl] + coff
                dstg[b][sl] = dstb[b][sl]

        def g_start(b):
            pltpu.async_copy(tab_hbm.at[gb[b]], rows[b], semg[b])

        def g_wait_scatter(b):
            pltpu.make_async_copy(tab_hbm.at[gb[b]], rows[b], semg[b]).wait()
            pltpu.sync_copy(rows[b], aggsp.at[dstg[b]], add=True)

        for k0 in range(NBUF):
            idx_start(k0, k0)
        for k0 in range(NBUF - 1):
            idx_wait(k0)
            compute_g(k0)
            g_start(k0)
            idx_start(k0 + NBUF, k0)

        def body(k4, _):
            for j in range(NBUF):
                b = (NBUF - 1 + j) % NBUF
                k = (NBUF - 1) + k4 * NBUF + j
                idx_wait(b)
                compute_g(b)
                g_start(b)
                idx_start(k + NBUF, b)
                g_wait_scatter((b + 1) % NBUF)
            return 0

        lax.fori_loop(0, (NCH - NBUF) // NBUF, body, 0)

        bl = (NCH - 1) % NBUF
        idx_wait(bl)
        compute_g(bl)
        g_start(bl)
        for d in range(NBUF):
            g_wait_scatter((bl + 1 + d) % NBUF)
        for d in range(NBUF - 1):
            idx_wait((bl + 1 + d) % NBUF)

        plsc.subcore_barrier()
        pltpu.sync_copy(aggsp.at[pl.ds(tbase, PW)],
                        out_hbm.at[c, pl.ds(tbase, PW)])

    return ek


@functools.lru_cache(maxsize=None)
def _sc_compact():
    ZW = NSP // 16

    @functools.partial(
        pl.kernel, mesh=_mesh(), compiler_params=_SC_PARAMS,
        out_type=jax.ShapeDtypeStruct((2, NSP), I32),
        scratch_types=[
            pltpu.VMEM((PW,), F32),
            pltpu.VMEM((128,), I32),
            pltpu.VMEM((128,), I32),
            pltpu.VMEM((16,), I32),
            pltpu.VMEM((256,), I32),
            pltpu.VMEM((ZW,), I32),
            pltpu.VMEM_SHARED((NSP,), I32),
            pltpu.VMEM_SHARED((256,), I32),
        ],
    )
    def ck(xu_hbm, xi_hbm, out_hbm,
           condb, posb, valb, cntb, callb, zbc, idxsp, cntsp):
        c = lax.axis_index("c")
        s = lax.axis_index("s")
        base = s * PW
        iota = lax.iota(I32, 16)

        @pl.when(c == 0)
        def _():
            pltpu.sync_copy(xu_hbm.at[pl.ds(base, PW)], condb)

        @pl.when(c == 1)
        def _():
            pltpu.sync_copy(xi_hbm.at[pl.ds(base, PW)], condb)

        def zfill(i, _):
            zbc[pl.ds(i * 16, 16)] = jnp.zeros((16,), I32)
            return 0
        lax.fori_loop(0, ZW // 16, zfill, 0)
        pltpu.sync_copy(zbc, idxsp.at[pl.ds(s * ZW, ZW)])

        one16 = jnp.ones((16,), I32)
        zero16 = jnp.zeros((16,), I32)

        def cnt(i, acc):
            f = condb[pl.ds(i * 16, 16)]
            return acc + jnp.sum(jnp.where(f != 0.0, one16, zero16))
        total = lax.fori_loop(0, PW // 16, cnt, jnp.zeros((), I32))
        cntb[...] = total + jnp.zeros((16,), I32)
        pltpu.sync_copy(cntb, cntsp.at[pl.ds(s * 16, 16)])
        plsc.subcore_barrier()

        pltpu.sync_copy(cntsp, callb)
        cvec = plsc.load_gather(callb, [iota * 16])
        excl0 = jnp.sum(jnp.where(iota < s, cvec, jnp.zeros((16,), I32)))

        def body(ko, off):
            for j in range(8):
                i = ko * 8 + j
                f = condb[pl.ds(i * 16, 16)]
                v = jnp.where(f != 0.0, one16, zero16)
                incl = plsc.cumsum(v)
                pos = off + incl - 1
                posm = jnp.where(v == 1, pos, NT + iota)
                sl = pl.ds(j * 16, 16)
                posb[sl] = posm
                valb[sl] = base + i * 16 + iota
                off = off + jnp.sum(v)
            pltpu.sync_copy(valb, idxsp.at[posb])
            return off

        lax.fori_loop(0, PW // 128, body, excl0)
        plsc.subcore_barrier()
        pltpu.sync_copy(idxsp.at[pl.ds(base, PW)], out_hbm.at[c, pl.ds(base, PW)])

        @pl.when(s == 15)
        def _():
            pltpu.sync_copy(idxsp.at[pl.ds(NT, 512)],
                            out_hbm.at[c, pl.ds(NT, 512)])

    return ck


@functools.lru_cache(maxsize=None)
def _sc_gather_ab():
    GC = 112
    NCHG = PW // GC
    NBG = 4

    @functools.partial(
        pl.kernel, mesh=_mesh(), compiler_params=_SC_PARAMS,
        out_type=jax.ShapeDtypeStruct((2, NT, 128), F32),
        scratch_types=[
            [pltpu.VMEM((GC,), I32)] * NBG,
            [pltpu.VMEM((GC,), I32)] * NBG,
            [pltpu.VMEM((GC, 128), F32)] * NBG,
            [pltpu.SemaphoreType.DMA] * NBG,
            [pltpu.SemaphoreType.DMA] * NBG,
        ],
    )
    def gk(uij_hbm, ab_hbm, out_hbm, idxb, gb, rows, semi, semg):
        c = lax.axis_index("c")
        s = lax.axis_index("s")
        base = s * PW
        coff = c * NT

        def idx_start(k, b):
            pltpu.async_copy(uij_hbm.at[c, pl.ds(base + k * GC, GC)],
                             idxb[b], semi[b])

        def idx_wait(b):
            pltpu.make_async_copy(uij_hbm.at[c, pl.ds(base, GC)],
                                  idxb[b], semi[b]).wait()

        def compute_g(b):
            for j in range(GC // 16):
                sl = pl.ds(j * 16, 16)
                gb[b][sl] = idxb[b][sl] + coff

        def g_start(b):
            pltpu.async_copy(ab_hbm.at[gb[b]], rows[b], semg[b])

        def g_wait_store(b, k):
            pltpu.make_async_copy(ab_hbm.at[gb[b]], rows[b], semg[b]).wait()
            pltpu.sync_copy(rows[b], out_hbm.at[c, pl.ds(base + k * GC, GC)])

        for k0 in range(NBG):
            idx_start(k0, k0)
        for k0 in range(NBG - 1):
            idx_wait(k0)
            compute_g(k0)
            g_start(k0)
            idx_start(k0 + NBG, k0)

        def body(k4, _):
            for j in range(NBG):
                b = (NBG - 1 + j) % NBG
                k = (NBG - 1) + k4 * NBG + j
                idx_wait(b)
                compute_g(b)
                g_start(b)
                idx_start(k + NBG, b)
                g_wait_store((b + 1) % NBG, k - NBG + 1)
            return 0

        lax.fori_loop(0, (NCHG - NBG) // NBG, body, 0)

        bl = (NCHG - 1) % NBG
        idx_wait(bl)
        compute_g(bl)
        g_start(bl)
        for d in range(NBG):
            bo = (bl + 1 + d) % NBG
            g_wait_store(bo, NCHG - NBG + d)
        for d in range(NBG - 1):
            idx_wait((bl + 1 + d) % NBG)

    return gk


def _wdiag(V, comp):
    din = V.shape[1]
    W3 = jnp.dot(comp, V.reshape(NB, din * 32)).reshape(R, din, 32)
    rows = []
    for r in range(R):
        rows.append(jnp.stack([
            block_diag(*([W3[r, :, h * 16:(h + 1) * 16]] * 8)) for h in (0, 1)
        ]))
    return jnp.stack(rows)


def kernel(x, edge_index, etype, V0, comp0, loop0, b0, V1, comp1, loop1, b1,
           V2, comp2, loop2, b2, V3, comp3, loop3, b3,
           lin1_w, lin1_b, lin2_w, lin2_b):
    npad = EPAD + NBUF * CH - E
    pad_src = jnp.zeros((1, npad), I32)
    pad_dst = jnp.full((1, npad), N, I32)
    eip = jnp.concatenate([edge_index, jnp.concatenate([pad_src, pad_dst], 0)], 1)
    etp = jnp.pad(etype, (0, npad))
    d1 = eip // 112
    rem = eip - d1 * 112
    off = jnp.stack([etp * NT, jnp.zeros_like(etp)])
    eip = (d1 >> 3) * 896 + rem * 8 + (d1 & 7) + off
    xu = jnp.pad(x[:, 0], (0, NT - N))
    xi = jnp.pad(x[:, 1], (0, NT - N))
    xp = jnp.pad(x, ((0, NT - N), (0, 0)))

    l0 = _tc_layer0()
    ll = _tc_layer()
    ek = _sc_edge()
    tab0, hl0 = l0(xp, _wdiag(V0, comp0), loop0, b0.reshape(1, 32))
    agg0 = ek(tab0.reshape(2 * R * NT, 16), eip).reshape(2, NT // 8, 128)
    s1, tab1, hl1 = ll(agg0, agg0, hl0, _wdiag(V1, comp1), loop1, b1.reshape(1, 32))
    agg1 = ek(tab1.reshape(2 * R * NT, 16), eip).reshape(2, NT // 8, 128)
    s2, tab2, hl2 = ll(agg1, agg1, hl1, _wdiag(V2, comp2), loop2, b2.reshape(1, 32))
    agg2 = ek(tab2.reshape(2 * R * NT, 16), eip).reshape(2, NT // 8, 128)
    s3, tab3, hl3 = ll(agg2, agg2, hl2, _wdiag(V3, comp3), loop3, b3.reshape(1, 32))
    agg3 = ek(tab3.reshape(2 * R * NT, 16), eip).reshape(2, NT // 8, 128)

    ab, = _tc_final_ab()(agg3, agg3, hl3, s1, s2, s3,
                         lin1_w[:128, :], lin1_w[128:, :])
    uij = _sc_compact()(xu, xi)
    s2g = _sc_gather_ab()(uij, ab.reshape(2 * NT, 128))
    o, = _tc_out()(s2g, s2g, lin1_b.reshape(1, 128), lin2_w, lin2_b.reshape(1, 1))
    return o[:N, 0]


# speedup vs baseline: 1.2786x; 1.0160x over previous
import functools

import jax
import jax.numpy as jnp
from jax import lax
from jax.scipy.linalg import block_diag
from jax.experimental import pallas as pl
from jax.experimental.pallas import tpu as pltpu
from jax.experimental.pallas import tpu_sc as plsc

N = 100000
E = 1600000
R = 5
NB = 2
NT = 100352
PW = 6272
CH = 128
NCH = 784
EPT = NCH * CH
EPAD = 16 * EPT
NBUF = 8
NSP = NT + 512
F32 = jnp.float32
I32 = jnp.int32


def _mesh():
    return plsc.VectorSubcoreMesh(core_axis_name="c", subcore_axis_name="s")


_SC_PARAMS = pltpu.CompilerParams(use_tc_tiling_on_sc=False,
                                  needs_layout_passes=False)


def _pack_rows(h, Bn):
    return jnp.concatenate([
        jnp.concatenate([h[g * 896 + q * 112: g * 896 + (q + 1) * 112]
                         for q in range(8)], axis=1)
        for g in range(Bn // 896)], axis=0)


def _unpack_cols(a, Bn):
    return jnp.concatenate([
        jnp.concatenate([a[g * 112:(g + 1) * 112, q * 16:(q + 1) * 16]
                         for q in range(8)], axis=0)
        for g in range(Bn // 896)], axis=0)


@functools.lru_cache(maxsize=None)
def _tc_layer0():
    Bn = 7168

    def body(x_ref, wd_ref, lw_ref, b_ref, tab_ref, hl_ref):
        h = x_ref[...]
        hp = _pack_rows(h, Bn)
        for r in range(R):
            tab_ref[0, r] = jnp.dot(hp, wd_ref[r, 0], preferred_element_type=F32)
            tab_ref[1, r] = jnp.dot(hp, wd_ref[r, 1], preferred_element_type=F32)
        hl_ref[...] = jnp.dot(h, lw_ref[...], preferred_element_type=F32) + b_ref[...]

    return pl.pallas_call(
        body, grid=(NT // Bn,),
        in_specs=[
            pl.BlockSpec((Bn, 4), lambda i: (i, 0)),
            pl.BlockSpec((R, 2, 32, 128), lambda i: (0, 0, 0, 0)),
            pl.BlockSpec((4, 32), lambda i: (0, 0)),
            pl.BlockSpec((1, 32), lambda i: (0, 0)),
        ],
        out_specs=[
            pl.BlockSpec((2, R, Bn // 8, 128), lambda i: (0, 0, i, 0)),
            pl.BlockSpec((Bn, 32), lambda i: (i, 0)),
        ],
        out_shape=[
            jax.ShapeDtypeStruct((2, R, NT // 8, 128), F32),
            jax.ShapeDtypeStruct((NT, 32), F32),
        ],
    )


@functools.lru_cache(maxsize=None)
def _tc_layer():
    Bn = 7168

    def body(alo_ref, ahi_ref, hlp_ref, wd_ref, lw_ref, b_ref,
             s_ref, tab_ref, hl_ref):
        alo = _unpack_cols(alo_ref[0], Bn)
        ahi = _unpack_cols(ahi_ref[0], Bn)
        agg = jnp.concatenate([alo, ahi], axis=-1)
        h = jnp.tanh(agg + hlp_ref[...])
        s_ref[...] = h
        hp = _pack_rows(h, Bn)
        for r in range(R):
            tab_ref[0, r] = jnp.dot(hp, wd_ref[r, 0], preferred_element_type=F32)
            tab_ref[1, r] = jnp.dot(hp, wd_ref[r, 1], preferred_element_type=F32)
        hl_ref[...] = jnp.dot(h, lw_ref[...], preferred_element_type=F32) + b_ref[...]

    return pl.pallas_call(
        body, grid=(NT // Bn,),
        in_specs=[
            pl.BlockSpec((1, Bn // 8, 128), lambda i: (0, i, 0)),
            pl.BlockSpec((1, Bn // 8, 128), lambda i: (1, i, 0)),
            pl.BlockSpec((Bn, 32), lambda i: (i, 0)),
            pl.BlockSpec((R, 2, 256, 128), lambda i: (0, 0, 0, 0)),
            pl.BlockSpec((32, 32), lambda i: (0, 0)),
            pl.BlockSpec((1, 32), lambda i: (0, 0)),
        ],
        out_specs=[
            pl.BlockSpec((Bn, 32), lambda i: (i, 0)),
            pl.BlockSpec((2, R, Bn // 8, 128), lambda i: (0, 0, i, 0)),
            pl.BlockSpec((Bn, 32), lambda i: (i, 0)),
        ],
        out_shape=[
            jax.ShapeDtypeStruct((NT, 32), F32),
            jax.ShapeDtypeStruct((2, R, NT // 8, 128), F32),
            jax.ShapeDtypeStruct((NT, 32), F32),
        ],
    )


@functools.lru_cache(maxsize=None)
def _tc_final_ab():
    Bn = 7168

    def body(alo_ref, ahi_ref, hl3_ref, s1_ref, s2_ref, s3_ref, wt_ref, wb_ref,
             ab_ref):
        alo = _unpack_cols(alo_ref[0], Bn)
        ahi = _unpack_cols(ahi_ref[0], Bn)
        agg = jnp.concatenate([alo, ahi], axis=-1)
        s4 = jnp.tanh(agg + hl3_ref[...])
        states = (s1_ref[...], s2_ref[...], s3_ref[...], s4)
        wt = wt_ref[...]
        wb = wb_ref[...]
        a = jnp.dot(states[0], wt[0:32, :], preferred_element_type=F32)
        b = jnp.dot(states[0], wb[0:32, :], preferred_element_type=F32)
        for k in range(1, 4):
            a = a + jnp.dot(states[k], wt[32 * k:32 * k + 32, :],
                            preferred_element_type=F32)
            b = b + jnp.dot(states[k], wb[32 * k:32 * k + 32, :],
                            preferred_element_type=F32)
        ab_ref[0] = a
        ab_ref[1] = b

    return pl.pallas_call(
        body, grid=(NT // Bn,),
        in_specs=[
            pl.BlockSpec((1, Bn // 8, 128), lambda i: (0, i, 0)),
            pl.BlockSpec((1, Bn // 8, 128), lambda i: (1, i, 0)),
            pl.BlockSpec((Bn, 32), lambda i: (i, 0)),
            pl.BlockSpec((Bn, 32), lambda i: (i, 0)),
            pl.BlockSpec((Bn, 32), lambda i: (i, 0)),
            pl.BlockSpec((Bn, 32), lambda i: (i, 0)),
            pl.BlockSpec((128, 128), lambda i: (0, 0)),
            pl.BlockSpec((128, 128), lambda i: (0, 0)),
        ],
        out_specs=[pl.BlockSpec((2, Bn, 128), lambda i: (0, i, 0))],
        out_shape=[jax.ShapeDtypeStruct((2, NT, 128), F32)],
    )


@functools.lru_cache(maxsize=None)
def _tc_out():
    Bn = 7168

    def body(su_ref, si_ref, b1_ref, w2_ref, b2_ref, o_ref):
        z = jnp.maximum(su_ref[0] + si_ref[0] + b1_ref[...], 0.0)
        o_ref[...] = jnp.dot(z, w2_ref[...], preferred_element_type=F32) + b2_ref[...]

    return pl.pallas_call(
        body, grid=(NT // Bn,),
        in_specs=[
            pl.BlockSpec((1, Bn, 128), lambda i: (0, i, 0)),
            pl.BlockSpec((1, Bn, 128), lambda i: (1, i, 0)),
            pl.BlockSpec((1, 128), lambda i: (0, 0)),
            pl.BlockSpec((128, 1), lambda i: (0, 0)),
            pl.BlockSpec((1, 1), lambda i: (0, 0)),
        ],
        out_specs=[pl.BlockSpec((Bn, 1), lambda i: (i, 0))],
        out_shape=[jax.ShapeDtypeStruct((NT, 1), F32)],
    )


@functools.lru_cache(maxsize=None)
def _sc_edge():
    @functools.partial(
        pl.kernel, mesh=_mesh(), compiler_params=_SC_PARAMS,
        out_type=jax.ShapeDtypeStruct((2, NT, 16), F32),
        scratch_types=[
            [pltpu.VMEM((CH,), I32)] * NBUF,
            [pltpu.VMEM((CH,), I32)] * NBUF,
            [pltpu.VMEM((CH,), I32)] * NBUF,
            [pltpu.VMEM((CH,), I32)] * NBUF,
            [pltpu.VMEM((CH, 16), F32)] * NBUF,
            pltpu.VMEM((112, 16), F32),
            pltpu.VMEM_SHARED((NT, 16), F32),
            [pltpu.SemaphoreType.DMA] * NBUF,
            [pltpu.SemaphoreType.DMA] * NBUF,
        ],
    )
    def ek(tab_hbm, ei_hbm, out_hbm,
           srcb, dstb, gb, dstg, rows, zb, aggsp, semi, semg):
        c = lax.axis_index("c")
        s = lax.axis_index("s")
        coff = c * (R * NT)
        tbase = s * PW

        def zfill(i, _):
            zb[i] = jnp.zeros((16,), F32)
            return 0
        lax.fori_loop(0, 112, zfill, 0)

        def zcopy(z, _):
            pltpu.sync_copy(zb, aggsp.at[pl.ds(tbase + z * 112, 112)])
            return 0
        lax.fori_loop(0, PW // 112, zcopy, 0)
        plsc.subcore_barrier()

        ebase = s * EPT

        def idx_start(k, b):
            base = ebase + k * CH
            pltpu.async_copy(ei_hbm.at[0, pl.ds(base, CH)], srcb[b], semi[b])
            pltpu.async_copy(ei_hbm.at[1, pl.ds(base, CH)], dstb[b], semi[b])

        def idx_wait(b):
            pltpu.make_async_copy(ei_hbm.at[0, pl.ds(0, CH)], srcb[b], semi[b]).wait()
            pltpu.make_async_copy(ei_hbm.at[0, pl.ds(0, CH)], dstb[b], semi[b]).wait()

        def compute_g(b):
            for j in range(CH // 16):
                sl = pl.ds(j * 16, 16)
                gb[b][sl] = srcb[b][sl] + coff
                dstg[b][sl] = dstb[b][sl]

        def g_start(b):
            pltpu.async_copy(tab_hbm.at[gb[b]], rows[b], semg[b])

        def g_wait_scatter(b):
            pltpu.make_async_copy(tab_hbm.at[gb[b]], rows[b], semg[b]).wait()
            pltpu.sync_copy(rows[b], aggsp.at[dstg[b]], add=True)

        for k0 in range(NBUF):
            idx_start(k0, k0)
        for k0 in range(NBUF - 1):
            idx_wait(k0)
            compute_g(k0)
            g_start(k0)
            idx_start(k0 + NBUF, k0)

        def body(k4, _):
            for j in range(NBUF):
                b = (NBUF - 1 + j) % NBUF
                k = (NBUF - 1) + k4 * NBUF + j
                idx_wait(b)
                compute_g(b)
                g_start(b)
                idx_start(k + NBUF, b)
                g_wait_scatter((b + 1) % NBUF)
            return 0

        lax.fori_loop(0, (NCH - NBUF) // NBUF, body, 0)

        bl = (NCH - 1) % NBUF
        idx_wait(bl)
        compute_g(bl)
        g_start(bl)
        for d in range(NBUF):
            g_wait_scatter((bl + 1 + d) % NBUF)
        for d in range(NBUF - 1):
            idx_wait((bl + 1 + d) % NBUF)

        plsc.subcore_barrier()
        pltpu.sync_copy(aggsp.at[pl.ds(tbase, PW)],
                        out_hbm.at[c, pl.ds(tbase, PW)])

    return ek


@functools.lru_cache(maxsize=None)
def _sc_compact():
    ZW = NSP // 16

    @functools.partial(
        pl.kernel, mesh=_mesh(), compiler_params=_SC_PARAMS,
        out_type=jax.ShapeDtypeStruct((2, NSP), I32),
        scratch_types=[
            pltpu.VMEM((PW,), F32),
            pltpu.VMEM((128,), I32),
            pltpu.VMEM((128,), I32),
            pltpu.VMEM((16,), I32),
            pltpu.VMEM((256,), I32),
            pltpu.VMEM((ZW,), I32),
            pltpu.VMEM_SHARED((NSP,), I32),
            pltpu.VMEM_SHARED((256,), I32),
        ],
    )
    def ck(xu_hbm, xi_hbm, out_hbm,
           condb, posb, valb, cntb, callb, zbc, idxsp, cntsp):
        c = lax.axis_index("c")
        s = lax.axis_index("s")
        base = s * PW
        iota = lax.iota(I32, 16)

        @pl.when(c == 0)
        def _():
            pltpu.sync_copy(xu_hbm.at[pl.ds(base, PW)], condb)

        @pl.when(c == 1)
        def _():
            pltpu.sync_copy(xi_hbm.at[pl.ds(base, PW)], condb)

        def zfill(i, _):
            zbc[pl.ds(i * 16, 16)] = jnp.zeros((16,), I32)
            return 0
        lax.fori_loop(0, ZW // 16, zfill, 0)
        pltpu.sync_copy(zbc, idxsp.at[pl.ds(s * ZW, ZW)])

        one16 = jnp.ones((16,), I32)
        zero16 = jnp.zeros((16,), I32)

        def cnt(i, acc):
            f = condb[pl.ds(i * 16, 16)]
            return acc + jnp.sum(jnp.where(f != 0.0, one16, zero16))
        total = lax.fori_loop(0, PW // 16, cnt, jnp.zeros((), I32))
        cntb[...] = total + jnp.zeros((16,), I32)
        pltpu.sync_copy(cntb, cntsp.at[pl.ds(s * 16, 16)])
        plsc.subcore_barrier()

        pltpu.sync_copy(cntsp, callb)
        cvec = plsc.load_gather(callb, [iota * 16])
        excl0 = jnp.sum(jnp.where(iota < s, cvec, jnp.zeros((16,), I32)))

        def body(ko, off):
            for j in range(8):
                i = ko * 8 + j
                f = condb[pl.ds(i * 16, 16)]
                v = jnp.where(f != 0.0, one16, zero16)
                incl = plsc.cumsum(v)
                pos = off + incl - 1
                posm = jnp.where(v == 1, pos, NT + iota)
                sl = pl.ds(j * 16, 16)
                posb[sl] = posm
                valb[sl] = base + i * 16 + iota
                off = off + jnp.sum(v)
            pltpu.sync_copy(valb, idxsp.at[posb])
            return off

        lax.fori_loop(0, PW // 128, body, excl0)
        plsc.subcore_barrier()
        pltpu.sync_copy(idxsp.at[pl.ds(base, PW)], out_hbm.at[c, pl.ds(base, PW)])

        @pl.when(s == 15)
        def _():
            pltpu.sync_copy(idxsp.at[pl.ds(NT, 512)],
                            out_hbm.at[c, pl.ds(NT, 512)])

    return ck


@functools.lru_cache(maxsize=None)
def _sc_gather_ab():
    GC = 112
    NCHG = PW // GC
    NBG = 4

    @functools.partial(
        pl.kernel, mesh=_mesh(), compiler_params=_SC_PARAMS,
        out_type=jax.ShapeDtypeStruct((2, NT, 128), F32),
        scratch_types=[
            [pltpu.VMEM((GC,), I32)] * NBG,
            [pltpu.VMEM((GC,), I32)] * NBG,
            [pltpu.VMEM((GC, 128), F32)] * NBG,
            [pltpu.SemaphoreType.DMA] * NBG,
            [pltpu.SemaphoreType.DMA] * NBG,
        ],
    )
    def gk(uij_hbm, ab_hbm, out_hbm, idxb, gb, rows, semi, semg):
        c = lax.axis_index("c")
        s = lax.axis_index("s")
        base = s * PW
        coff = c * NT

        def idx_start(k, b):
            pltpu.async_copy(uij_hbm.at[c, pl.ds(base + k * GC, GC)],
                             idxb[b], semi[b])

        def idx_wait(b):
            pltpu.make_async_copy(uij_hbm.at[c, pl.ds(base, GC)],
                                  idxb[b], semi[b]).wait()

        def compute_g(b):
            for j in range(GC // 16):
                sl = pl.ds(j * 16, 16)
                gb[b][sl] = idxb[b][sl] + coff

        def g_start(b):
            pltpu.async_copy(ab_hbm.at[gb[b]], rows[b], semg[b])

        def g_wait_store(b, k):
            pltpu.make_async_copy(ab_hbm.at[gb[b]], rows[b], semg[b]).wait()
            pltpu.sync_copy(rows[b], out_hbm.at[c, pl.ds(base + k * GC, GC)])

        for k0 in range(NBG):
            idx_start(k0, k0)
        for k0 in range(NBG - 1):
            idx_wait(k0)
            compute_g(k0)
            g_start(k0)
            idx_start(k0 + NBG, k0)

        def body(k4, _):
            for j in range(NBG):
                b = (NBG - 1 + j) % NBG
                k = (NBG - 1) + k4 * NBG + j
                idx_wait(b)
                compute_g(b)
                g_start(b)
                idx_start(k + NBG, b)
                g_wait_store((b + 1) % NBG, k - NBG + 1)
            return 0

        lax.fori_loop(0, (NCHG - NBG) // NBG, body, 0)

        bl = (NCHG - 1) % NBG
        idx_wait(bl)
        compute_g(bl)
        g_start(bl)
        for d in range(NBG):
            bo = (bl + 1 + d) % NBG
            g_wait_store(bo, NCHG - NBG + d)
        for d in range(NBG - 1):
            idx_wait((bl + 1 + d) % NBG)

    return gk


def _wdiag(V, comp):
    din = V.shape[1]
    W3 = jnp.dot(comp, V.reshape(NB, din * 32)).reshape(R, din, 32)
    rows = []
    for r in range(R):
        rows.append(jnp.stack([
            block_diag(*([W3[r, :, h * 16:(h + 1) * 16]] * 8)) for h in (0, 1)
        ]))
    return jnp.stack(rows)


def kernel(x, edge_index, etype, V0, comp0, loop0, b0, V1, comp1, loop1, b1,
           V2, comp2, loop2, b2, V3, comp3, loop3, b3,
           lin1_w, lin1_b, lin2_w, lin2_b):
    npad = EPAD + NBUF * CH - E
    pad_src = jnp.zeros((1, npad), I32)
    pad_dst = jnp.full((1, npad), N, I32)
    eip = jnp.concatenate([edge_index, jnp.concatenate([pad_src, pad_dst], 0)], 1)
    etp = jnp.pad(etype, (0, npad))
    d1 = eip // 112
    rem = eip - d1 * 112
    off = jnp.stack([etp * NT, jnp.zeros_like(etp)])
    eip = (d1 >> 3) * 896 + rem * 8 + (d1 & 7) + off
    xu = jnp.pad(x[:, 0], (0, NT - N))
    xi = jnp.pad(x[:, 1], (0, NT - N))
    xp = jnp.pad(x, ((0, NT - N), (0, 0)))

    l0 = _tc_layer0()
    ll = _tc_layer()
    ek = _sc_edge()
    tab0, hl0 = l0(xp, _wdiag(V0, comp0), loop0, b0.reshape(1, 32))
    agg0 = ek(tab0.reshape(2 * R * NT, 16), eip).reshape(2, NT // 8, 128)
    s1, tab1, hl1 = ll(agg0, agg0, hl0, _wdiag(V1, comp1), loop1, b1.reshape(1, 32))
    agg1 = ek(tab1.reshape(2 * R * NT, 16), eip).reshape(2, NT // 8, 128)
    s2, tab2, hl2 = ll(agg1, agg1, hl1, _wdiag(V2, comp2), loop2, b2.reshape(1, 32))
    agg2 = ek(tab2.reshape(2 * R * NT, 16), eip).reshape(2, NT // 8, 128)
    s3, tab3, hl3 = ll(agg2, agg2, hl2, _wdiag(V3, comp3), loop3, b3.reshape(1, 32))
    agg3 = ek(tab3.reshape(2 * R * NT, 16), eip).reshape(2, NT // 8, 128)

    ab, = _tc_final_ab()(agg3, agg3, hl3, s1, s2, s3,
                         lin1_w[:128, :], lin1_w[128:, :])
    uij = _sc_compact()(xu, xi)
    s2g = _sc_gather_ab()(uij, ab.reshape(2 * NT, 128))
    o, = _tc_out()(s2g, s2g, lin1_b.reshape(1, 128), lin2_w, lin2_b.reshape(1, 1))
    return o[:N, 0]
